# baseline (speedup 1.0000x reference)
import functools

import jax
import jax.numpy as jnp
from jax import lax
from jax.experimental import pallas as pl
from jax.experimental.pallas import tpu as pltpu
from jax.experimental.pallas import tpu_sc as plsc

TOPK = 16
LANES = 16
NEG_INF = float("-inf")
BB = 8
NW = 32
SPAD = 128


def _scores_body(lite_ref, scores_ref):
    s = jnp.max(lite_ref[...], axis=2)
    pad = jnp.zeros((BB, SPAD - s.shape[1]), jnp.float32)
    scores_ref[...] = jnp.concatenate([s, pad], axis=1)


def _xor_reduce(v, op):
    iota = lax.broadcasted_iota(jnp.int32, (LANES,), 0)
    for s in (8, 4, 2, 1):
        v = op(v, v.at[iota ^ s].get(mode="promise_in_bounds"))
    return v


def _sc_topk_gather_mean_body(
    scores_hbm, in_hbm, out_hbm, sc_v, rows_v, idx_v, out_v, gsem0, gsem1,
    *, bpw, base_b, T,
):
    D = in_hbm.shape[1]
    nc = 2
    wid = lax.axis_index("s") * nc + lax.axis_index("c")
    iota = lax.broadcasted_iota(jnp.int32, (LANES,), 0)
    pltpu.sync_copy(scores_hbm.at[pl.ds(wid * bpw, bpw)], sc_v)

    gsems = [gsem0, gsem1]
    big = jnp.int32(2**30)
    for bb in range(bpw):
        svecs = [sc_v[bb, pl.ds(j * LANES, LANES)] for j in range(4)]
        idx_acc = jnp.zeros((LANES,), jnp.int32)
        for k in range(TOPK):
            m = jnp.maximum(
                jnp.maximum(svecs[0], svecs[1]), jnp.maximum(svecs[2], svecs[3])
            )
            mx = _xor_reduce(m, jnp.maximum)
            cands = [
                jnp.where(svecs[j] == mx, iota + j * LANES, big) for j in range(4)
            ]
            cmin = jnp.minimum(
                jnp.minimum(cands[0], cands[1]), jnp.minimum(cands[2], cands[3])
            )
            t = _xor_reduce(cmin, jnp.minimum)
            idx_acc = jnp.where(
                iota == k, (base_b + wid * bpw + bb) * T + t, idx_acc
            )
            for j in range(4):
                svecs[j] = jnp.where(iota + j * LANES == t, NEG_INF, svecs[j])
        idx_v[bb, :] = idx_acc
        pltpu.make_async_copy(
            in_hbm.at[idx_v.at[bb]], rows_v.at[bb], gsems[bb]
        ).start()

    for bb in range(bpw):
        pltpu.make_async_copy(
            in_hbm.at[idx_v.at[bb]], rows_v.at[bb], gsems[bb]
        ).wait()

        @pl.loop(0, D // LANES)
        def _mean(cidx):
            sl = pl.ds(cidx * LANES, LANES)
            acc = rows_v[bb, 0, sl]
            for r in range(1, TOPK):
                acc = acc + rows_v[bb, r, sl]
            out_v[bb, sl] = acc * (1.0 / TOPK)

    pltpu.sync_copy(out_v, out_hbm.at[pl.ds(wid * bpw, bpw)])


@jax.jit
def kernel(input, lite_input):
    B, T, D = input.shape
    half = B // 2
    bpw = half // NW

    def scores_call(off_blocks):
        return pl.pallas_call(
            _scores_body,
            grid=(half // BB,),
            in_specs=[
                pl.BlockSpec((BB, T, D), lambda b: (b + off_blocks, 0, 0))
            ],
            out_specs=pl.BlockSpec((BB, SPAD), lambda b: (b, 0)),
            out_shape=jax.ShapeDtypeStruct((half, SPAD), jnp.float32),
        )(lite_input)

    input_rows = input.reshape(B * T, D)

    def sc_call(base_b):
        return pl.kernel(
            functools.partial(
                _sc_topk_gather_mean_body, bpw=bpw, base_b=base_b, T=T
            ),
            out_type=jax.ShapeDtypeStruct((half, D), jnp.float32),
            mesh=plsc.VectorSubcoreMesh(
                core_axis_name="c", subcore_axis_name="s"
            ),
            scratch_types=[
                pltpu.VMEM((bpw, SPAD), jnp.float32),
                pltpu.VMEM((bpw, TOPK, D), jnp.float32),
                pltpu.VMEM((bpw, TOPK), jnp.int32),
                pltpu.VMEM((bpw, D), jnp.float32),
                pltpu.SemaphoreType.DMA,
                pltpu.SemaphoreType.DMA,
            ],
        )

    s0 = scores_call(0)
    o0 = sc_call(0)(s0, input_rows)
    s1 = scores_call(half // BB)
    o1 = sc_call(half)(s1, input_rows)

    out = jnp.concatenate([o0, o1], axis=0)
    return out.reshape(B, 1, D)


# speedup vs baseline: 1.1288x; 1.1288x over previous
import jax
import jax.numpy as jnp
from jax import lax
from jax.experimental import pallas as pl
from jax.experimental.pallas import tpu as pltpu
from jax.experimental.pallas import tpu_sc as plsc

TOPK = 16
LANES = 16
NEG_INF = float("-inf")
BB = 8
BPW = 2
SPAD = 128


def _scores_body(lite_ref, scores_ref):
    s = jnp.max(lite_ref[...], axis=2)
    pad = jnp.zeros((BB, SPAD - s.shape[1]), jnp.float32)
    scores_ref[...] = jnp.concatenate([s, pad], axis=1)


def _xor_reduce(v, op):
    iota = lax.broadcasted_iota(jnp.int32, (LANES,), 0)
    for s in (8, 4, 2, 1):
        v = op(v, v.at[iota ^ s].get(mode="promise_in_bounds"))
    return v


def _sc_topk_gather_mean_body(
    scores_hbm, in_hbm, out_hbm, sc_v, rows_v, idx_v, out_v, gsem0, gsem1
):
    D = in_hbm.shape[1]
    T = 64
    nc = 2
    wid = lax.axis_index("s") * nc + lax.axis_index("c")
    iota = lax.broadcasted_iota(jnp.int32, (LANES,), 0)
    pltpu.sync_copy(scores_hbm.at[pl.ds(wid * BPW, BPW)], sc_v)

    gsems = [gsem0, gsem1]
    big = jnp.int32(2**30)
    for bb in range(BPW):
        svecs = [sc_v[bb, pl.ds(j * LANES, LANES)] for j in range(4)]
        idx_acc = jnp.zeros((LANES,), jnp.int32)
        for k in range(TOPK):
            m = jnp.maximum(
                jnp.maximum(svecs[0], svecs[1]), jnp.maximum(svecs[2], svecs[3])
            )
            mx = _xor_reduce(m, jnp.maximum)
            cands = [
                jnp.where(svecs[j] == mx, iota + j * LANES, big) for j in range(4)
            ]
            cmin = jnp.minimum(
                jnp.minimum(cands[0], cands[1]), jnp.minimum(cands[2], cands[3])
            )
            t = _xor_reduce(cmin, jnp.minimum)
            idx_acc = jnp.where(iota == k, (wid * BPW + bb) * T + t, idx_acc)
            for j in range(4):
                svecs[j] = jnp.where(iota + j * LANES == t, NEG_INF, svecs[j])
        idx_v[bb, :] = idx_acc
        pltpu.make_async_copy(
            in_hbm.at[idx_v.at[bb]], rows_v.at[bb], gsems[bb]
        ).start()

    for bb in range(BPW):
        pltpu.make_async_copy(
            in_hbm.at[idx_v.at[bb]], rows_v.at[bb], gsems[bb]
        ).wait()

        @pl.loop(0, D // (2 * LANES))
        def _mean(cidx):
            for h in range(2):
                sl = pl.ds(cidx * 2 * LANES + h * LANES, LANES)
                acc = rows_v[bb, 0, sl]
                for r in range(1, TOPK):
                    acc = acc + rows_v[bb, r, sl]
                out_v[bb, 0, sl] = acc * (1.0 / TOPK)

    pltpu.sync_copy(out_v, out_hbm.at[pl.ds(wid * BPW, BPW)])


@jax.jit
def kernel(input, lite_input):
    B, T, D = input.shape

    scores = pl.pallas_call(
        _scores_body,
        grid=(B // BB,),
        in_specs=[pl.BlockSpec((BB, T, D), lambda b: (b, 0, 0))],
        out_specs=pl.BlockSpec((BB, SPAD), lambda b: (b, 0)),
        out_shape=jax.ShapeDtypeStruct((B, SPAD), jnp.float32),
    )(lite_input)

    input_rows = input.reshape(B * T, D)

    sc_stage = pl.kernel(
        _sc_topk_gather_mean_body,
        out_type=jax.ShapeDtypeStruct((B, 1, D), jnp.float32),
        mesh=plsc.VectorSubcoreMesh(core_axis_name="c", subcore_axis_name="s"),
        scratch_types=[
            pltpu.VMEM((BPW, SPAD), jnp.float32),
            pltpu.VMEM((BPW, TOPK, D), jnp.float32),
            pltpu.VMEM((BPW, TOPK), jnp.int32),
            pltpu.VMEM((BPW, 1, D), jnp.float32),
            pltpu.SemaphoreType.DMA,
            pltpu.SemaphoreType.DMA,
        ],
    )
    return sc_stage(scores, input_rows)


# speedup vs baseline: 1.1457x; 1.0150x over previous
import jax
import jax.numpy as jnp
from jax import lax
from jax.experimental import pallas as pl
from jax.experimental.pallas import tpu as pltpu
from jax.experimental.pallas import tpu_sc as plsc

TOPK = 16
LANES = 16
NEG_INF = float("-inf")
BB = 16
BPW = 2
SPAD = 128


def _scores_body(lite_ref, scores_ref):
    s = jnp.max(lite_ref[...], axis=2)
    pad = jnp.zeros((BB, SPAD - s.shape[1]), jnp.float32)
    scores_ref[...] = jnp.concatenate([s, pad], axis=1)


def _xor_reduce(v, op):
    iota = lax.broadcasted_iota(jnp.int32, (LANES,), 0)
    for s in (8, 4, 2, 1):
        v = op(v, v.at[iota ^ s].get(mode="promise_in_bounds"))
    return v


def _sc_topk_gather_mean_body(
    scores_hbm, in_hbm, out_hbm, sc_v, rows_v, idx_v, out_v, gsem0, gsem1
):
    D = in_hbm.shape[1]
    T = 64
    nc = 2
    wid = lax.axis_index("s") * nc + lax.axis_index("c")
    iota = lax.broadcasted_iota(jnp.int32, (LANES,), 0)
    pltpu.sync_copy(scores_hbm.at[pl.ds(wid * BPW, BPW)], sc_v)

    gsems = [gsem0, gsem1]
    big = jnp.int32(2**30)
    for bb in range(BPW):
        svecs = [sc_v[bb, pl.ds(j * LANES, LANES)] for j in range(4)]
        idx_acc = jnp.zeros((LANES,), jnp.int32)
        for k in range(TOPK):
            m = jnp.maximum(
                jnp.maximum(svecs[0], svecs[1]), jnp.maximum(svecs[2], svecs[3])
            )
            mx = _xor_reduce(m, jnp.maximum)
            cands = [
                jnp.where(svecs[j] == mx, iota + j * LANES, big) for j in range(4)
            ]
            cmin = jnp.minimum(
                jnp.minimum(cands[0], cands[1]), jnp.minimum(cands[2], cands[3])
            )
            t = _xor_reduce(cmin, jnp.minimum)
            idx_acc = jnp.where(iota == k, (wid * BPW + bb) * T + t, idx_acc)
            for j in range(4):
                svecs[j] = jnp.where(iota + j * LANES == t, NEG_INF, svecs[j])
        idx_v[bb, :] = idx_acc
        pltpu.make_async_copy(
            in_hbm.at[idx_v.at[bb]], rows_v.at[bb], gsems[bb]
        ).start()

    for bb in range(BPW):
        pltpu.make_async_copy(
            in_hbm.at[idx_v.at[bb]], rows_v.at[bb], gsems[bb]
        ).wait()

        @pl.loop(0, D // (2 * LANES))
        def _mean(cidx):
            for h in range(2):
                sl = pl.ds(cidx * 2 * LANES + h * LANES, LANES)
                acc = rows_v[bb, 0, sl]
                for r in range(1, TOPK):
                    acc = acc + rows_v[bb, r, sl]
                out_v[bb, 0, sl] = acc * (1.0 / TOPK)

    pltpu.sync_copy(out_v, out_hbm.at[pl.ds(wid * BPW, BPW)])


@jax.jit
def kernel(input, lite_input):
    B, T, D = input.shape

    scores = pl.pallas_call(
        _scores_body,
        grid=(B // BB,),
        in_specs=[pl.BlockSpec((BB, T, D), lambda b: (b, 0, 0))],
        out_specs=pl.BlockSpec((BB, SPAD), lambda b: (b, 0)),
        out_shape=jax.ShapeDtypeStruct((B, SPAD), jnp.float32),
    )(lite_input)

    input_rows = input.reshape(B * T, D)

    sc_stage = pl.kernel(
        _sc_topk_gather_mean_body,
        out_type=jax.ShapeDtypeStruct((B, 1, D), jnp.float32),
        mesh=plsc.VectorSubcoreMesh(core_axis_name="c", subcore_axis_name="s"),
        scratch_types=[
            pltpu.VMEM((BPW, SPAD), jnp.float32),
            pltpu.VMEM((BPW, TOPK, D), jnp.float32),
            pltpu.VMEM((BPW, TOPK), jnp.int32),
            pltpu.VMEM((BPW, 1, D), jnp.float32),
            pltpu.SemaphoreType.DMA,
            pltpu.SemaphoreType.DMA,
        ],
    )
    return sc_stage(scores, input_rows)
